# baseline (speedup 1.0000x reference)
import functools

import jax
import jax.numpy as jnp
from jax import lax
from jax.experimental import pallas as pl
from jax.experimental.pallas import tpu as pltpu
from jax.experimental.pallas import tpu_sc as plsc

R = 16
N = 1_000_000
NW = 32
HALF = N * R // NW
BS = 2000
NB = HALF // BS
BROW = 512
CH = 10_000
NCH = HALF // CH
ETA = 0.1

_mesh = plsc.VectorSubcoreMesh(core_axis_name="c", subcore_axis_name="s")


def _wid():
    return lax.axis_index("s") * 2 + lax.axis_index("c")


def _splat_f(x):
    return jnp.full((16,), x, jnp.float32)


def _splat_i(x):
    return jnp.full((16,), x, jnp.int32)


@functools.partial(
    pl.kernel,
    out_type=jax.ShapeDtypeStruct((R * BROW,), jnp.float32),
    mesh=_mesh,
    compiler_params=pltpu.CompilerParams(needs_layout_passes=False),
    scratch_types=[
        pltpu.VMEM((CH,), jnp.float32),
        pltpu.VMEM((256,), jnp.float32),
    ],
)
def _phase_a(w, bs_out, buf, bsv):
    wid = _wid()
    base = wid * HALF
    r = wid // 2
    h = wid % 2
    zero = jnp.zeros((16,), jnp.float32)
    for i in range(16):
        bsv[pl.ds(16 * i, 16)] = zero
    iota = lax.iota(jnp.int32, 16)
    lane0 = iota == 0

    def chunk(k, _):
        pltpu.sync_copy(w.at[pl.ds(base + k * CH, CH)], buf)
        for b in range(CH // BS):
            accs = [zero, zero, zero, zero]
            for j in range(BS // 16):
                accs[j % 4] = accs[j % 4] + buf[pl.ds(b * BS + j * 16, 16)]
            acc = (accs[0] + accs[1]) + (accs[2] + accs[3])
            s = jnp.sum(acc)
            pos = k * (CH // BS) + b
            plsc.store_scatter(bsv, [_splat_i(pos)], _splat_f(s), mask=lane0)
        return 0

    lax.fori_loop(0, NCH, chunk, 0)
    pltpu.sync_copy(bsv, bs_out.at[pl.ds(r * BROW + h * 256, 256)])


@functools.partial(
    pl.kernel,
    out_type=(
        jax.ShapeDtypeStruct((R * 16,), jnp.int32),
        jax.ShapeDtypeStruct((R * 16,), jnp.float32),
        jax.ShapeDtypeStruct((R * 16,), jnp.float32),
    ),
    mesh=_mesh,
    compiler_params=pltpu.CompilerParams(needs_layout_passes=False),
    scratch_types=[
        pltpu.VMEM((BROW,), jnp.float32),
        pltpu.VMEM((BROW,), jnp.float32),
        pltpu.VMEM((BS,), jnp.float32),
        pltpu.VMEM((16,), jnp.float32),
        pltpu.VMEM((16,), jnp.float32),
        pltpu.VMEM((16,), jnp.int32),
        pltpu.VMEM((16,), jnp.float32),
        pltpu.VMEM((16,), jnp.float32),
    ],
)
def _phase_b(w, bs, draw, loss, choice_out, nw_out, sc_out,
             bsv, cumv, blk, drv, lsv, oi, of1, of2):
    wid = _wid()
    BIG = jnp.int32(2**30)
    iota = lax.iota(jnp.int32, 16)

    @pl.when(wid < R)
    def _():
        r = wid
        pltpu.sync_copy(bs.at[pl.ds(r * BROW, BROW)], bsv)
        pltpu.sync_copy(draw, drv)
        pltpu.sync_copy(loss, lsv)
        carry = jnp.float32(0.0)
        for i in range(BROW // 16):
            v = bsv[pl.ds(16 * i, 16)]
            c = plsc.cumsum(v) + carry
            cumv[pl.ds(16 * i, 16)] = c
            carry = jnp.max(c)
        S = carry
        d = jnp.max(plsc.load_gather(drv, [_splat_i(r)]))
        t = d * S
        best = BIG
        for i in range(BROW // 16):
            c = cumv[pl.ds(16 * i, 16)]
            cand = jnp.where(c >= t, iota + 16 * i, BIG)
            best = jnp.minimum(best, jnp.min(cand))
        bstar = jnp.minimum(best, jnp.int32(256 + NB - 1))
        pidx = jnp.maximum(bstar - 1, 0)
        pv = plsc.load_gather(cumv, [_splat_i(pidx)])
        prefix = jnp.where(bstar == 0, jnp.float32(0.0), jnp.max(pv))
        h = (bstar >= 256).astype(jnp.int32)
        jb = bstar - h * 256
        off = pl.multiple_of(r * N + h * HALF + jb * BS, 8)
        pltpu.sync_copy(w.at[pl.ds(off, BS)], blk)
        tp = t - prefix
        carry2 = jnp.float32(0.0)
        best2 = BIG
        for i in range(BS // 16):
            v = blk[pl.ds(16 * i, 16)]
            c = plsc.cumsum(v) + carry2
            cand = jnp.where(c >= tp, iota + 16 * i, BIG)
            best2 = jnp.minimum(best2, jnp.min(cand))
            carry2 = jnp.max(c)
        jloc = jnp.minimum(best2, jnp.int32(BS - 1))
        choice = h * HALF + jb * BS + jloc
        wc_v = plsc.load_gather(blk, [_splat_i(jloc)])
        l_v = plsc.load_gather(lsv, [_splat_i(r)])
        S_v = _splat_f(S)
        p_c = wc_v / S_v
        est = l_v / jnp.maximum(p_c, _splat_f(1e-12))
        f = jnp.exp(_splat_f(-ETA) * est)
        nwc_v = wc_v * f
        scale_v = _splat_f(1.0) / (S_v - wc_v + nwc_v)
        oi[...] = _splat_i(choice)
        of1[...] = nwc_v
        of2[...] = scale_v
        pltpu.sync_copy(oi, choice_out.at[pl.ds(r * 16, 16)])
        pltpu.sync_copy(of1, nw_out.at[pl.ds(r * 16, 16)])
        pltpu.sync_copy(of2, sc_out.at[pl.ds(r * 16, 16)])


@functools.partial(
    pl.kernel,
    out_type=(
        jax.ShapeDtypeStruct((R * N,), jnp.float32),
        jax.ShapeDtypeStruct((R * N,), jnp.float32),
    ),
    mesh=_mesh,
    compiler_params=pltpu.CompilerParams(needs_layout_passes=False),
    scratch_types=[
        pltpu.VMEM((CH,), jnp.float32),
        pltpu.VMEM((CH,), jnp.float32),
        pltpu.VMEM((16,), jnp.int32),
        pltpu.VMEM((16,), jnp.float32),
        pltpu.VMEM((16,), jnp.float32),
    ],
)
def _phase_c(w, choice, nwc, scl, nw_out, np_out, buf, pbuf, civ, nwv, scv):
    wid = _wid()
    base = wid * HALF
    r = wid // 2
    h = wid % 2
    pltpu.sync_copy(choice.at[pl.ds(r * 16, 16)], civ)
    pltpu.sync_copy(nwc.at[pl.ds(r * 16, 16)], nwv)
    pltpu.sync_copy(scl.at[pl.ds(r * 16, 16)], scv)
    ci = jnp.max(civ[...])
    nw_vec = nwv[...]
    sc_vec = scv[...]
    cl = ci - h * HALF
    mine = (cl >= 0) & (cl < HALF)
    iota = lax.iota(jnp.int32, 16)
    lane0 = iota == 0

    def chunk(k, _):
        off = base + k * CH
        pltpu.sync_copy(w.at[pl.ds(off, CH)], buf)
        lo = cl - k * CH

        @pl.when(mine & (lo >= 0) & (lo < CH))
        def _():
            plsc.store_scatter(buf, [_splat_i(lo)], nw_vec, mask=lane0)

        for j in range(CH // 16):
            pbuf[pl.ds(j * 16, 16)] = buf[pl.ds(j * 16, 16)] * sc_vec
        pltpu.sync_copy(buf, nw_out.at[pl.ds(off, CH)])
        pltpu.sync_copy(pbuf, np_out.at[pl.ds(off, CH)])
        return 0

    lax.fori_loop(0, NCH, chunk, 0)


def kernel(weights, draw, loss):
    wf = weights.reshape(-1)
    bs = _phase_a(wf)
    choice16, nwc16, scl16 = _phase_b(wf, bs, draw, loss)
    nw, np_ = _phase_c(wf, choice16, nwc16, scl16)
    choice = choice16.reshape(R, 16)[:, 0]
    return choice, nw.reshape(R, N), np_.reshape(R, N)


# speedup vs baseline: 6.3015x; 6.3015x over previous
import jax
import jax.numpy as jnp
from jax import lax
from jax.experimental import pallas as pl
from jax.experimental.pallas import tpu as pltpu

R = 16
N = 1_000_000
CK = 8000
NCK = N // CK
NCH = R * NCK
SUB = 8
GRID = NCH // SUB
ETA = 0.1


def _csum_body(w_ref, out_ref):
    out_ref[...] = jnp.sum(w_ref[...], axis=1)[None, None, :]


_csum = pl.pallas_call(
    _csum_body,
    grid=(GRID,),
    in_specs=[pl.BlockSpec((SUB, CK), lambda k: (k, 0))],
    out_specs=pl.BlockSpec((1, 1, SUB), lambda k: (k, 0, 0)),
    out_shape=jax.ShapeDtypeStruct((GRID, 1, SUB), jnp.float32),
)


def _cumsum1(x):
    n = x.shape[1]
    sh = 1
    while sh < n:
        pad = jnp.zeros((x.shape[0], sh), x.dtype)
        x = x + jnp.concatenate([pad, x[:, : n - sh]], axis=1)
        sh *= 2
    return x


def _first_true(m, fallback):
    n = m.shape[1]
    ii = lax.broadcasted_iota(jnp.int32, m.shape, 1)
    idx = jnp.min(jnp.where(m, ii, n), axis=1)
    return jnp.where(idx == n, fallback, idx).astype(jnp.int32)


def _at(x, idx):
    ii = lax.broadcasted_iota(jnp.int32, x.shape, 1)
    return jnp.sum(jnp.where(ii == idx[:, None], x, 0.0), axis=1)


def _coarse_body(cs_ref, draw_ref, choice_ref, tp_ref, s_ref):
    cs = cs_ref[...]
    cum = _cumsum1(cs)
    S = cum[:, NCK - 1]
    t = draw_ref[...] * S
    bidx = _first_true(cum >= t[:, None], NCK - 1)
    prefix = _at(cum, bidx) - _at(cs, bidx)
    choice_ref[...] = bidx
    tp_ref[...] = t - prefix
    s_ref[...] = S


_coarse = pl.pallas_call(
    _coarse_body,
    in_specs=[
        pl.BlockSpec(memory_space=pltpu.MemorySpace.VMEM),
        pl.BlockSpec(memory_space=pltpu.MemorySpace.VMEM),
    ],
    out_specs=(
        pl.BlockSpec(memory_space=pltpu.MemorySpace.VMEM),
        pl.BlockSpec(memory_space=pltpu.MemorySpace.VMEM),
        pl.BlockSpec(memory_space=pltpu.MemorySpace.VMEM),
    ),
    out_shape=(
        jax.ShapeDtypeStruct((R,), jnp.int32),
        jax.ShapeDtypeStruct((R,), jnp.float32),
        jax.ShapeDtypeStruct((R,), jnp.float32),
    ),
)

FSUB = 8
FLANE = CK // FSUB


def _extract(vec_ref, r):
    v = vec_ref[...]
    return jnp.sum(jnp.where(lax.iota(jnp.int32, R) == r, v, 0 * v))


def _fine_body(bidx_ref, w_ref, tp_ref, s_ref, loss_ref,
               choice_ref, nwc_ref, scale_ref):
    r = pl.program_id(0)
    tp = _extract(tp_ref, r)
    S = _extract(s_ref, r)
    l = _extract(loss_ref, r)
    bidx = bidx_ref[r]
    chunk = w_ref[0, 0]
    rs = jnp.sum(chunk, axis=1)
    rcum = _cumsum1(rs[None, :])[0]
    ii = lax.iota(jnp.int32, FSUB)
    istar = jnp.min(jnp.where(rcum >= tp, ii, FSUB))
    istar = jnp.minimum(istar, FSUB - 1)
    pre_i = jnp.sum(jnp.where(ii == istar, rcum - rs, 0.0))
    ii0 = lax.broadcasted_iota(jnp.int32, (FSUB, FLANE), 0)
    rowvec = jnp.sum(jnp.where(ii0 == istar, chunk, 0.0), axis=0)
    rc = _cumsum1(rowvec[None, :])[0]
    jj = lax.iota(jnp.int32, FLANE)
    jstar = jnp.min(jnp.where(rc >= tp - pre_i, jj, FLANE))
    jstar = jnp.minimum(jstar, FLANE - 1)
    wc = jnp.sum(jnp.where(jj == jstar, rowvec, 0.0))
    choice = bidx * CK + istar * FLANE + jstar
    wc_v = jnp.full((1, FSUB), wc)
    S_v = jnp.full((1, FSUB), S)
    est = jnp.full((1, FSUB), l) / jnp.maximum(wc_v / S_v, 1e-12)
    f = jnp.exp(-ETA * est)
    nwc_v = wc_v * f
    scale_v = 1.0 / (S_v - wc_v + nwc_v)
    choice_ref[...] = jnp.full((1, 1, FSUB), choice, jnp.int32)
    nwc_ref[...] = nwc_v[None]
    scale_ref[...] = scale_v[None]


_osp = pl.BlockSpec((1, 1, FSUB), lambda r, b: (r, 0, 0))
_vsp = pl.BlockSpec(memory_space=pltpu.MemorySpace.VMEM)
_fine = pl.pallas_call(
    _fine_body,
    grid_spec=pltpu.PrefetchScalarGridSpec(
        num_scalar_prefetch=1,
        grid=(R,),
        in_specs=[
            pl.BlockSpec((1, 1, FSUB, FLANE), lambda r, b: (r, b[r], 0, 0)),
            _vsp,
            _vsp,
            _vsp,
        ],
        out_specs=(_osp, _osp, _osp),
    ),
    out_shape=(
        jax.ShapeDtypeStruct((R, 1, FSUB), jnp.int32),
        jax.ShapeDtypeStruct((R, 1, FSUB), jnp.float32),
        jax.ShapeDtypeStruct((R, 1, FSUB), jnp.float32),
    ),
)


def _apply_body(w_ref, cloc_ref, nwc_ref, scale_ref, nw_ref, np_ref):
    cloc = cloc_ref[0, 0, :]
    nwc = nwc_ref[0, 0, :]
    scale = scale_ref[0, 0, :]
    j = lax.broadcasted_iota(jnp.int32, (SUB, CK), 1)
    m = j == cloc[:, None]
    nw = jnp.where(m, nwc[:, None], w_ref[...])
    nw_ref[...] = nw
    np_ref[...] = nw * scale[:, None]


_scal_spec = pl.BlockSpec((1, 1, SUB), lambda k: (k, 0, 0))
_apply = pl.pallas_call(
    _apply_body,
    grid=(GRID,),
    in_specs=[
        pl.BlockSpec((SUB, CK), lambda k: (k, 0)),
        _scal_spec,
        _scal_spec,
        _scal_spec,
    ],
    out_specs=(
        pl.BlockSpec((SUB, CK), lambda k: (k, 0)),
        pl.BlockSpec((SUB, CK), lambda k: (k, 0)),
    ),
    out_shape=(
        jax.ShapeDtypeStruct((NCH, CK), jnp.float32),
        jax.ShapeDtypeStruct((NCH, CK), jnp.float32),
    ),
)


def kernel(weights, draw, loss):
    wf = weights.reshape(NCH, CK)
    cs = _csum(wf)
    bidx, tp, S = _coarse(cs.reshape(R, NCK), draw)
    choice3, nwc3, scale3 = _fine(bidx, weights.reshape(R, NCK, FSUB, FLANE),
                                  tp, S, loss)
    choice, nwc, scale = choice3[:, 0, 0], nwc3[:, 0, 0], scale3[:, 0, 0]
    chunk_col = jnp.arange(NCH, dtype=jnp.int32) % NCK * CK
    choice_rep = jnp.repeat(choice, NCK)
    cloc = jnp.where((choice_rep >= chunk_col) & (choice_rep < chunk_col + CK),
                     choice_rep - chunk_col, -1)
    nw, np_ = _apply(wf, cloc.reshape(GRID, 1, SUB),
                     jnp.repeat(nwc, NCK).reshape(GRID, 1, SUB),
                     jnp.repeat(scale, NCK).reshape(GRID, 1, SUB))
    return choice, nw.reshape(R, N), np_.reshape(R, N)


# speedup vs baseline: 43.3100x; 6.8730x over previous
import jax
import jax.numpy as jnp
from jax import lax
from jax.experimental import pallas as pl
from jax.experimental.pallas import tpu as pltpu

R = 16
N = 1_000_000
BK = 41600
NBK = 24
MAIN = NBK * BK
TAIL = N - MAIN
NSEG = BK // 128
ETA = 0.1


def _cumsum1(x):
    n = x.shape[1]
    sh = 1
    while sh < n:
        pad = jnp.zeros((x.shape[0], sh), x.dtype)
        x = x + jnp.concatenate([pad, x[:, : n - sh]], axis=1)
        sh *= 2
    return x


def _first_true(m, fallback):
    n = m.shape[1]
    ii = lax.broadcasted_iota(jnp.int32, m.shape, 1)
    idx = jnp.min(jnp.where(m, ii, n), axis=1)
    return jnp.where(idx == n, fallback, idx).astype(jnp.int32)


def _at(x, idx):
    ii = lax.broadcasted_iota(jnp.int32, x.shape, 1)
    return jnp.sum(jnp.where(ii == idx[:, None], x, 0.0), axis=1)


def _body(w_hbm, draw_ref, loss_ref,
          choice_ref, nw_hbm, np_hbm,
          wbuf, mainbuf, tailbuf, nwbuf, npbuf, ntw, ntp,
          insem, tsem, fsem, outsem, tosem):

    def start_in(k):
        pltpu.make_async_copy(
            w_hbm.at[:, pl.ds(k * BK, BK)], wbuf.at[k % 2], insem.at[k % 2]
        ).start()

    def wait_in(k):
        pltpu.make_async_copy(
            w_hbm.at[:, pl.ds(k * BK, BK)], wbuf.at[k % 2], insem.at[k % 2]
        ).wait()

    tail_cp = pltpu.make_async_copy(
        w_hbm.at[:, pl.ds(MAIN, TAIL)], tailbuf, tsem)
    tail_cp.start()
    start_in(0)
    cs_parts = []
    for k in range(NBK):
        if k + 1 < NBK:
            start_in(k + 1)
        wait_in(k)
        cs_parts.append(jnp.sum(wbuf[k % 2], axis=1, keepdims=True))
    tail_cp.wait()
    tail = tailbuf[...]
    cs_parts.append(jnp.sum(tail, axis=1, keepdims=True))
    cs = jnp.concatenate(cs_parts, axis=1)

    cum = _cumsum1(cs)
    S = cum[:, NBK]
    t = draw_ref[...] * S
    bidx = _first_true(cum >= t[:, None], NBK)
    prefix = _at(cum, bidx) - _at(cs, bidx)
    tp = t - prefix
    is_tail = bidx == NBK

    rr = lax.iota(jnp.int32, R)
    bclamp = jnp.minimum(bidx, NBK - 1)
    fetches = []
    for r in range(R):
        br = jnp.sum(jnp.where(rr == r, bclamp, 0))
        cp = pltpu.make_async_copy(
            w_hbm.at[pl.ds(r, 1), pl.ds(br * BK, BK)],
            mainbuf.at[pl.ds(r, 1), :],
            fsem.at[r],
        )
        cp.start()
        fetches.append(cp)
    for cp in fetches:
        cp.wait()

    blk = mainbuf[...]
    seg3 = blk.reshape(R, NSEG, 128)
    segs = jnp.sum(seg3, axis=2)
    segcum = _cumsum1(segs)
    sstar = _first_true(segcum >= tp[:, None], NSEG - 1)
    spre = _at(segcum, sstar) - _at(segs, sstar)
    ii1 = lax.broadcasted_iota(jnp.int32, (R, NSEG, 128), 1)
    seg = jnp.sum(jnp.where(ii1 == sstar[:, None, None], seg3, 0.0), axis=1)
    segc = _cumsum1(seg)
    tp2 = tp - spre
    jstar = _first_true(segc >= tp2[:, None], 127)
    wc_m = _at(seg, jstar)
    choice_m = bclamp * BK + sstar * 128 + jstar

    tailc = _cumsum1(tail)
    jt = _first_true(tailc >= tp[:, None], TAIL - 1)
    wc_t = _at(tail, jt)
    choice_t = MAIN + jt

    choice = jnp.where(is_tail, choice_t, choice_m)
    wc = jnp.where(is_tail, wc_t, wc_m)
    est = loss_ref[...] / jnp.maximum(wc / S, 1e-12)
    f = jnp.exp(-ETA * est)
    nwc = wc * f
    scale = 1.0 / (S - wc + nwc)
    choice_ref[...] = choice

    def start_out(k):
        pltpu.make_async_copy(
            nwbuf.at[k % 2], nw_hbm.at[:, pl.ds(k * BK, BK)],
            outsem.at[k % 2, 0]).start()
        pltpu.make_async_copy(
            npbuf.at[k % 2], np_hbm.at[:, pl.ds(k * BK, BK)],
            outsem.at[k % 2, 1]).start()

    def wait_out(k):
        pltpu.make_async_copy(
            nwbuf.at[k % 2], nw_hbm.at[:, pl.ds(k * BK, BK)],
            outsem.at[k % 2, 0]).wait()
        pltpu.make_async_copy(
            npbuf.at[k % 2], np_hbm.at[:, pl.ds(k * BK, BK)],
            outsem.at[k % 2, 1]).wait()

    jtail = lax.broadcasted_iota(jnp.int32, (R, TAIL), 1) + MAIN
    nw_tail = jnp.where(jtail == choice[:, None], nwc[:, None], tail)
    ntw[...] = nw_tail
    ntp[...] = nw_tail * scale[:, None]
    tw_cp = pltpu.make_async_copy(ntw, nw_hbm.at[:, pl.ds(MAIN, TAIL)],
                                  tosem.at[0])
    tp_cp = pltpu.make_async_copy(ntp, np_hbm.at[:, pl.ds(MAIN, TAIL)],
                                  tosem.at[1])
    tw_cp.start()
    tp_cp.start()

    start_in(0)
    for k in range(NBK):
        if k + 1 < NBK:
            start_in(k + 1)
        wait_in(k)
        w = wbuf[k % 2]
        j = lax.broadcasted_iota(jnp.int32, (R, BK), 1) + k * BK
        nw = jnp.where(j == choice[:, None], nwc[:, None], w)
        if k >= 2:
            wait_out(k - 2)
        nwbuf[k % 2] = nw
        npbuf[k % 2] = nw * scale[:, None]
        start_out(k)
    wait_out(NBK - 2)
    wait_out(NBK - 1)
    tw_cp.wait()
    tp_cp.wait()


_hbm = pl.BlockSpec(memory_space=pltpu.MemorySpace.HBM)
_vmem = pl.BlockSpec(memory_space=pltpu.MemorySpace.VMEM)

_fused = pl.pallas_call(
    _body,
    in_specs=[_hbm, _vmem, _vmem],
    out_specs=(_vmem, _hbm, _hbm),
    out_shape=(
        jax.ShapeDtypeStruct((R,), jnp.int32),
        jax.ShapeDtypeStruct((R, N), jnp.float32),
        jax.ShapeDtypeStruct((R, N), jnp.float32),
    ),
    scratch_shapes=[
        pltpu.VMEM((2, R, BK), jnp.float32),
        pltpu.VMEM((R, BK), jnp.float32),
        pltpu.VMEM((R, TAIL), jnp.float32),
        pltpu.VMEM((2, R, BK), jnp.float32),
        pltpu.VMEM((2, R, BK), jnp.float32),
        pltpu.VMEM((R, TAIL), jnp.float32),
        pltpu.VMEM((R, TAIL), jnp.float32),
        pltpu.SemaphoreType.DMA((2,)),
        pltpu.SemaphoreType.DMA,
        pltpu.SemaphoreType.DMA((R,)),
        pltpu.SemaphoreType.DMA((2, 2)),
        pltpu.SemaphoreType.DMA((2,)),
    ],
)


def kernel(weights, draw, loss):
    choice, nw, np_ = _fused(weights, draw, loss)
    return choice, nw, np_
